# baseline (speedup 1.0000x reference)
import functools

import jax
import jax.numpy as jnp
from jax import lax
from jax.experimental import pallas as pl
from jax.experimental.pallas import tpu as pltpu
from jax.experimental.pallas import tpu_sc as plsc

_STYLE_NUM = 100
_FRAME_NUM = 1000
_LATENT_DIM = 64
_B = 16384

_NC = 2
_NS = 16
_NW = _NC * _NS
_CHQ = 2048
_L = 16

_SENT = 127 << 10


def _gather_body(style_hbm, frame_hbm, lat_hbm, out_hbm,
                 sbuf, fbuf, ml, bl, plane_v, rows16, psem, osem):
    wid = lax.axis_index("s") * _NC + lax.axis_index("c")
    lo = wid * _STYLE_NUM // _NW
    hi = (wid + 1) * _STYLE_NUM // _NW

    iota = lax.iota(jnp.int32, _L)
    didx = [iota + k * _L for k in range(_LATENT_DIM // _L)]
    sent16 = jnp.full((_L,), _SENT, jnp.int32)

    def scan_chunk(q, cnt):
        pltpu.sync_copy(style_hbm.at[pl.ds(q * _CHQ, _CHQ)], sbuf)
        pltpu.sync_copy(frame_hbm.at[pl.ds(q * _CHQ, _CHQ)], fbuf)

        def scan_vreg(g, c):
            s16 = sbuf[pl.ds(g * _L, _L)]
            f16 = fbuf[pl.ds(g * _L, _L)]
            posv = jnp.full((_L,), q * _CHQ + g * _L, jnp.int32) + iota
            mask = (s16 >= lo) & (s16 < hi)
            v = posv * 131072 + s16 * 1024 + f16
            plsc.store_compressed(ml.at[pl.ds(c, _L)], v, mask=mask)
            return c + plsc.all_reduce_population_count(mask)[0]

        return lax.fori_loop(0, _CHQ // _L, scan_vreg, cnt)

    cnt = lax.fori_loop(0, _B // _CHQ, scan_chunk, jnp.int32(0))
    ml[pl.ds(cnt, _L)] = sent16
    n_mvreg = (cnt + _L - 1) // _L

    def style_step(b, carry):
        cnt2, gblk = carry
        s = lo + b
        cp = pltpu.async_copy(lat_hbm.at[s], plane_v, psem)

        def bucket_vreg(m, c):
            v16 = ml[pl.ds(m * _L, _L)]
            mask = ((v16 >> 10) & 127) == s
            plsc.store_compressed(bl.at[pl.ds(c, _L)], v16, mask=mask)
            return c + plsc.all_reduce_population_count(mask)[0]

        c_end = lax.fori_loop(0, n_mvreg, bucket_vreg, cnt2)
        last_v = bl[pl.ds(jnp.maximum(c_end - 1, 0), _L)][0]
        bl[pl.ds(c_end, _L)] = lax.broadcast(last_v, (_L,))
        nblk = (c_end - cnt2 + _L - 1) // _L
        cp.wait()

        def block(t, gb):
            @pl.when(gb >= 2)
            def _():
                pltpu.make_async_copy(
                    rows16.at[0], out_hbm.at[pl.ds(0, _L)], osem).wait()

            buf = gb % 2
            blk = bl[pl.ds(cnt2 + t * _L, _L)]
            for i in range(_L):
                v = blk[i]
                f = v & 1023
                pos = lax.shift_right_logical(v, 17)
                fidx = lax.broadcast(f, (_L,))
                for k in range(_LATENT_DIM // _L):
                    rows16[buf, i, pl.ds(k * _L, _L)] = plsc.load_gather(
                        plane_v, [didx[k], fidx])
                pltpu.async_copy(rows16.at[buf, i], out_hbm.at[pos], osem)
            return gb + 1

        gblk = lax.fori_loop(0, nblk, block, gblk)
        return cnt2 + nblk * _L, gblk

    _, gblk = lax.fori_loop(0, hi - lo, style_step,
                            (jnp.int32(0), jnp.int32(0)))

    def final_drain(r, c):
        pltpu.make_async_copy(
            rows16.at[0], out_hbm.at[pl.ds(0, _L)], osem).wait()
        return c

    lax.fori_loop(0, jnp.minimum(gblk, 2), final_drain, 0)


@jax.jit
def _sc_gather(style_ids, frame_ids, lat_t):
    mesh = plsc.VectorSubcoreMesh(core_axis_name="c", subcore_axis_name="s")
    return pl.kernel(
        _gather_body,
        out_type=jax.ShapeDtypeStruct((_B, _LATENT_DIM), jnp.float32),
        mesh=mesh,
        scratch_types=[
            pltpu.VMEM((_CHQ,), jnp.int32),
            pltpu.VMEM((_CHQ,), jnp.int32),
            pltpu.VMEM((_B + _L,), jnp.int32),
            pltpu.VMEM((_B + 8 * _L,), jnp.int32),
            pltpu.VMEM((_LATENT_DIM, _FRAME_NUM), jnp.float32),
            pltpu.VMEM((2, _L, _LATENT_DIM), jnp.float32),
            pltpu.SemaphoreType.DMA,
            pltpu.SemaphoreType.DMA,
        ],
        compiler_params=pltpu.CompilerParams(
            use_tc_tiling_on_sc=True, needs_layout_passes=False),
    )(style_ids, frame_ids, lat_t)


def kernel(style_ids, frame_ids, type, latents, style_latents_mu):
    del type, style_latents_mu
    return _sc_gather(style_ids, frame_ids,
                      jnp.transpose(latents, (0, 2, 1)))


# speedup vs baseline: 1.6261x; 1.6261x over previous
import functools

import jax
import jax.numpy as jnp
from jax import lax
from jax.experimental import pallas as pl
from jax.experimental.pallas import tpu as pltpu
from jax.experimental.pallas import tpu_sc as plsc

_STYLE_NUM = 100
_FRAME_NUM = 1000
_LATENT_DIM = 64
_B = 16384

_NC = 2
_NS = 16
_NW = _NC * _NS
_BPW = _B // _NW
_CH = 64


def _gather_body(style_hbm, frame_hbm, lat_hbm, out_hbm,
                 sid_v, fid_v, flat_v, rows_v, sem, osem):
    wid = lax.axis_index("s") * _NC + lax.axis_index("c")
    base = wid * _BPW

    pltpu.sync_copy(style_hbm.at[pl.ds(base, _BPW)], sid_v)
    pltpu.sync_copy(frame_hbm.at[pl.ds(base, _BPW)], fid_v)

    def ids(g, c):
        s = sid_v[pl.ds(g * 16, 16)]
        f = fid_v[pl.ds(g * 16, 16)]
        flat_v[pl.ds(g * 16, 16)] = s * _FRAME_NUM + f
        return c

    lax.fori_loop(0, _BPW // 16, ids, 0)

    def fire_chunk(j):
        def fire_group(g, c):
            k0 = j * _CH + g * 16
            r16 = flat_v[pl.ds(k0, 16)]
            for i in range(16):
                pltpu.async_copy(lat_hbm.at[r16[i]], rows_v.at[k0 + i], sem)
            return c

        lax.fori_loop(0, _CH // 16, fire_group, 0)

    def drain_chunk(j):
        pltpu.make_async_copy(
            lat_hbm.at[pl.ds(0, _CH)],
            rows_v.at[pl.ds(j * _CH, _CH)], sem).wait()
        pltpu.async_copy(rows_v.at[pl.ds(j * _CH, _CH)],
                         out_hbm.at[pl.ds(base + j * _CH, _CH)], osem)

    fire_chunk(0)

    def step(j, carry):
        fire_chunk(j + 1)
        drain_chunk(j)
        return carry

    lax.fori_loop(0, _BPW // _CH - 1, step, 0)
    drain_chunk(_BPW // _CH - 1)

    pltpu.make_async_copy(rows_v, out_hbm.at[pl.ds(base, _BPW)], osem).wait()


@jax.jit
def _sc_gather(style_ids, frame_ids, lat_flat):
    mesh = plsc.VectorSubcoreMesh(core_axis_name="c", subcore_axis_name="s")
    return pl.kernel(
        _gather_body,
        out_type=jax.ShapeDtypeStruct((_B, _LATENT_DIM), jnp.float32),
        mesh=mesh,
        scratch_types=[
            pltpu.VMEM((_BPW,), jnp.int32),
            pltpu.VMEM((_BPW,), jnp.int32),
            pltpu.VMEM((_BPW,), jnp.int32),
            pltpu.VMEM((_BPW, _LATENT_DIM), jnp.float32),
            pltpu.SemaphoreType.DMA,
            pltpu.SemaphoreType.DMA,
        ],
        compiler_params=pltpu.CompilerParams(use_tc_tiling_on_sc=True),
    )(style_ids, frame_ids, lat_flat)


def kernel(style_ids, frame_ids, type, latents, style_latents_mu):
    del type, style_latents_mu
    return _sc_gather(style_ids, frame_ids, latents.reshape(-1, _LATENT_DIM))
